# Initial kernel scaffold; baseline (speedup 1.0000x reference)
import jax
import jax.numpy as jnp
from jax.experimental import pallas as pl


def kernel(nodes_u, nodes_v, hist_u_n, hist_u_r, adj_u, hist_v_n, hist_v_r, adj_v, u2e, v2e, r2e, relation_att, Wq, bq, W1, b1):
    raise NotImplementedError("write your pallas kernel here")


# speedup vs baseline: 9.6864x; 9.6864x over previous
import functools

import jax
import jax.numpy as jnp
from jax import lax
from jax.experimental import pallas as pl
from jax.experimental.pallas import tpu as pltpu
from jax.experimental.pallas import tpu_sc as plsc

NU = 100000
NI = 100000
D = 64
B = 4096
L = 50
M = 50
T = L + M
K = T // 2

_NC = 2
_NS = 16
_NW = _NC * _NS

_BIG_PER_W = (B * L) // _NW
_CHUNK = 800
_SMALL_PER_W = B // _NW


def _sc_gather_kernel(u2e, v2e, nu_i, nv_i, hun_i, au_i, hvn_i, av_i,
                      a_out, b_out, huh_out, hua_out, hvh_out, hva_out,
                      idx_big, rows_big, idx_sml, rows_sml, sem):
    wid = lax.axis_index("s") * _NC + lax.axis_index("c")

    def gather_big(idx_hbm, table_hbm, out_hbm):
        base = wid * _BIG_PER_W
        for i in range(_BIG_PER_W // _CHUNK):
            off = base + i * _CHUNK
            pltpu.sync_copy(idx_hbm.at[pl.ds(off, _CHUNK)], idx_big)
            pltpu.async_copy(table_hbm.at[idx_big], rows_big, sem).wait()
            pltpu.sync_copy(rows_big, out_hbm.at[pl.ds(off, _CHUNK)])

    def gather_small(idx_hbm, table_hbm, out_hbm):
        off = wid * _SMALL_PER_W
        pltpu.sync_copy(idx_hbm.at[pl.ds(off, _SMALL_PER_W)], idx_sml)
        pltpu.async_copy(table_hbm.at[idx_sml], rows_sml, sem).wait()
        pltpu.sync_copy(rows_sml, out_hbm.at[pl.ds(off, _SMALL_PER_W)])

    gather_small(nu_i, u2e, a_out)
    gather_small(nv_i, v2e, b_out)
    gather_big(hun_i, v2e, huh_out)
    gather_big(au_i, u2e, hua_out)
    gather_big(hvn_i, u2e, hvh_out)
    gather_big(av_i, v2e, hva_out)


def _sc_gather(u2e, v2e, nodes_u, nodes_v, hist_u_n, adj_u, hist_v_n, adj_v):
    mesh = plsc.VectorSubcoreMesh(core_axis_name="c", subcore_axis_name="s")
    f32 = jnp.float32
    out_type = [
        jax.ShapeDtypeStruct((B, D), f32),
        jax.ShapeDtypeStruct((B, D), f32),
        jax.ShapeDtypeStruct((B * L, D), f32),
        jax.ShapeDtypeStruct((B * L, D), f32),
        jax.ShapeDtypeStruct((B * L, D), f32),
        jax.ShapeDtypeStruct((B * L, D), f32),
    ]
    run = functools.partial(
        pl.kernel, mesh=mesh, out_type=out_type,
        compiler_params=pltpu.CompilerParams(use_tc_tiling_on_sc=False),
        scratch_types=[
            pltpu.VMEM((_CHUNK,), jnp.int32),
            pltpu.VMEM((_CHUNK, D), f32),
            pltpu.VMEM((_SMALL_PER_W,), jnp.int32),
            pltpu.VMEM((_SMALL_PER_W, D), f32),
            pltpu.SemaphoreType.DMA,
        ],
    )(_sc_gather_kernel)
    return run(u2e, v2e, nodes_u, nodes_v,
               hist_u_n.reshape(B * L), adj_u.reshape(B * L),
               hist_v_n.reshape(B * L), adj_v.reshape(B * L))


_BT = 128


def _tc_body(a_ref, b_ref, huh_ref, hua_ref, hvh_ref, hva_ref,
             gu_ref, gv_ref, ru_ref, rv_ref,
             att_ref, wq_ref, bq_ref, w1_ref, b1_ref, r2e_ref, out_ref):
    att1 = att_ref[0, :D]
    att2 = att_ref[0, D:]
    s_r = jnp.sum(r2e_ref[...] * att2[None, :], axis=1)
    tri = (lax.broadcasted_iota(jnp.int32, (T, T), 1)
           < lax.broadcasted_iota(jnp.int32, (T, T), 0))

    def encode(nf, tf, hf, g, r_idx):
        q = jnp.concatenate([nf, tf], axis=-1) @ wq_ref[...] + bq_ref[0, :]
        diff = q[:, None, :] - hf
        dist = jnp.sqrt(jnp.sum(diff * diff, axis=-1))
        y = g - dist
        ygt = (y[:, None, :] > y[:, :, None]).astype(jnp.float32)
        yeq = jnp.logical_and(y[:, None, :] == y[:, :, None], tri[None, :, :])
        rank = jnp.sum(ygt + yeq.astype(jnp.float32), axis=2)
        mask = rank < K
        s_rel = jnp.zeros_like(y)
        for r in range(7):
            s_rel = s_rel + jnp.where(r_idx == r, s_r[r], 0.0)
        sc = jnp.sum(hf * att1[None, None, :], axis=-1) + s_rel
        scm = jnp.where(mask, sc, -jnp.inf)
        mx = jnp.max(scm, axis=1, keepdims=True)
        p = jnp.where(mask, jnp.exp(scm - mx), 0.0)
        p2 = p / jnp.sum(p, axis=1, keepdims=True)
        emb = jnp.sum(hf * p2[:, :, None], axis=1)
        comb = jnp.concatenate([nf, emb], axis=-1) @ w1_ref[...] + b1_ref[0, :]
        return jnp.maximum(comb, 0.0)

    a = a_ref[...]
    b = b_ref[...]
    hfu = jnp.concatenate([huh_ref[...], hua_ref[...]], axis=1)
    hfv = jnp.concatenate([hvh_ref[...], hva_ref[...]], axis=1)
    eu = encode(a, b, hfu, gu_ref[...], ru_ref[...])
    ev = encode(b, a, hfv, gv_ref[...], rv_ref[...])
    out_ref[...] = jnp.sum(eu * ev, axis=1, keepdims=True)


def _tc_scores(a, b, huh, hua, hvh, hva, gu, gv, ru, rv,
               att, wq, bq, w1, b1, r2e, interpret=False):
    nb = B // _BT
    bspec = lambda blk, imap: pl.BlockSpec(blk, imap)
    grid_spec = pl.GridSpec(
        grid=(nb,),
        in_specs=[
            bspec((_BT, D), lambda i: (i, 0)),
            bspec((_BT, D), lambda i: (i, 0)),
            bspec((_BT, L, D), lambda i: (i, 0, 0)),
            bspec((_BT, L, D), lambda i: (i, 0, 0)),
            bspec((_BT, L, D), lambda i: (i, 0, 0)),
            bspec((_BT, L, D), lambda i: (i, 0, 0)),
            bspec((_BT, T), lambda i: (i, 0)),
            bspec((_BT, T), lambda i: (i, 0)),
            bspec((_BT, T), lambda i: (i, 0)),
            bspec((_BT, T), lambda i: (i, 0)),
            bspec((1, 2 * D), lambda i: (0, 0)),
            bspec((2 * D, D), lambda i: (0, 0)),
            bspec((1, D), lambda i: (0, 0)),
            bspec((2 * D, D), lambda i: (0, 0)),
            bspec((1, D), lambda i: (0, 0)),
            bspec((8, D), lambda i: (0, 0)),
        ],
        out_specs=bspec((_BT, 1), lambda i: (i, 0)),
    )
    out = pl.pallas_call(
        _tc_body,
        grid_spec=grid_spec,
        out_shape=jax.ShapeDtypeStruct((B, 1), jnp.float32),
        interpret=interpret,
    )(a, b, huh, hua, hvh, hva, gu, gv, ru, rv, att, wq, bq, w1, b1, r2e)
    return out.reshape(B)


def _gumbel(key, shape):
    u = jax.random.uniform(key, shape)
    return -jnp.log(-jnp.log(u + 1e-10) + 1e-10)


def kernel(nodes_u, nodes_v, hist_u_n, hist_u_r, adj_u, hist_v_n, hist_v_r,
           adj_v, u2e, v2e, r2e, relation_att, Wq, bq, W1, b1):
    i32 = jnp.int32
    nodes_u = nodes_u.astype(i32)
    nodes_v = nodes_v.astype(i32)
    hist_u_n = hist_u_n.astype(i32)
    adj_u = adj_u.astype(i32)
    hist_v_n = hist_v_n.astype(i32)
    adj_v = adj_v.astype(i32)

    a, b, huh, hua, hvh, hva = _sc_gather(
        u2e, v2e, nodes_u, nodes_v, hist_u_n, adj_u, hist_v_n, adj_v)

    gu = _gumbel(jax.random.fold_in(jax.random.key(42), 1), (B, T))
    gv = _gumbel(jax.random.fold_in(jax.random.key(42), 2), (B, T))
    six = jnp.full((B, M), 6, dtype=i32)
    ru = jnp.concatenate([hist_u_r.astype(i32), six], axis=1)
    rv = jnp.concatenate([hist_v_r.astype(i32), six], axis=1)
    r2e_p = jnp.concatenate([r2e, jnp.zeros((1, D), r2e.dtype)], axis=0)

    return _tc_scores(
        a, b,
        huh.reshape(B, L, D), hua.reshape(B, L, D),
        hvh.reshape(B, L, D), hva.reshape(B, L, D),
        gu, gv, ru, rv,
        relation_att.reshape(1, 2 * D), Wq, bq.reshape(1, D),
        W1, b1.reshape(1, D), r2e_p)


# speedup vs baseline: 19.4715x; 2.0102x over previous
import functools

import jax
import jax.numpy as jnp
from jax import lax
from jax.experimental import pallas as pl
from jax.experimental.pallas import tpu as pltpu
from jax.experimental.pallas import tpu_sc as plsc

NU = 100000
NI = 100000
D = 64
B = 4096
L = 50
M = 50
T = L + M
K = T // 2

_NC = 2
_NS = 16
_NW = _NC * _NS

_BIG_PER_W = (B * L) // _NW
_CHUNK = 800
_SMALL_PER_W = B // _NW


def _sc_gather_kernel(u2e, v2e, nu_i, nv_i, hun_i, au_i, hvn_i, av_i,
                      a_out, b_out, huh_out, hua_out, hvh_out, hva_out,
                      idx_big, rows_big, idx_sml, rows_sml, sem):
    wid = lax.axis_index("s") * _NC + lax.axis_index("c")

    def gather_big(idx_hbm, table_hbm, out_hbm):
        base = wid * _BIG_PER_W
        for i in range(_BIG_PER_W // _CHUNK):
            off = base + i * _CHUNK
            pltpu.sync_copy(idx_hbm.at[pl.ds(off, _CHUNK)], idx_big)
            pltpu.async_copy(table_hbm.at[idx_big], rows_big, sem).wait()
            pltpu.sync_copy(rows_big, out_hbm.at[pl.ds(off, _CHUNK)])

    def gather_small(idx_hbm, table_hbm, out_hbm):
        off = wid * _SMALL_PER_W
        pltpu.sync_copy(idx_hbm.at[pl.ds(off, _SMALL_PER_W)], idx_sml)
        pltpu.async_copy(table_hbm.at[idx_sml], rows_sml, sem).wait()
        pltpu.sync_copy(rows_sml, out_hbm.at[pl.ds(off, _SMALL_PER_W)])

    gather_small(nu_i, u2e, a_out)
    gather_small(nv_i, v2e, b_out)
    gather_big(hun_i, v2e, huh_out)
    gather_big(au_i, u2e, hua_out)
    gather_big(hvn_i, u2e, hvh_out)
    gather_big(av_i, v2e, hva_out)


def _sc_gather(u2e, v2e, nodes_u, nodes_v, hist_u_n, adj_u, hist_v_n, adj_v):
    mesh = plsc.VectorSubcoreMesh(core_axis_name="c", subcore_axis_name="s")
    f32 = jnp.float32
    out_type = [
        jax.ShapeDtypeStruct((B, D), f32),
        jax.ShapeDtypeStruct((B, D), f32),
        jax.ShapeDtypeStruct((B * L, D), f32),
        jax.ShapeDtypeStruct((B * L, D), f32),
        jax.ShapeDtypeStruct((B * L, D), f32),
        jax.ShapeDtypeStruct((B * L, D), f32),
    ]
    run = functools.partial(
        pl.kernel, mesh=mesh, out_type=out_type,
        compiler_params=pltpu.CompilerParams(use_tc_tiling_on_sc=False),
        scratch_types=[
            pltpu.VMEM((_CHUNK,), jnp.int32),
            pltpu.VMEM((_CHUNK, D), f32),
            pltpu.VMEM((_SMALL_PER_W,), jnp.int32),
            pltpu.VMEM((_SMALL_PER_W, D), f32),
            pltpu.SemaphoreType.DMA,
        ],
    )(_sc_gather_kernel)
    return run(u2e, v2e, nodes_u, nodes_v,
               hist_u_n.reshape(B * L), adj_u.reshape(B * L),
               hist_v_n.reshape(B * L), adj_v.reshape(B * L))


_BT = 128


def _tc_body(a_ref, b_ref, huh_ref, hua_ref, hvh_ref, hva_ref,
             gu_ref, gv_ref, ru_ref, rv_ref,
             att_ref, wqt_ref, bq_ref, w1t_ref, b1_ref, r2e_ref, out_ref):
    att1T = att_ref[0:1, :D].T
    att2 = att_ref[0:1, D:]
    tri = (lax.broadcasted_iota(jnp.int32, (T, T), 1)
           <= lax.broadcasted_iota(jnp.int32, (T, T), 0)).astype(jnp.float32)

    def encode(nfT, tfT, hfT_h, hfT_a, gT, rT):
        catT = jnp.concatenate([nfT, tfT], axis=0)
        qT = wqt_ref[...] @ catT + bq_ref[...]

        def half_stats(hfT):
            diff = qT[:, None, :] - hfT
            d2 = jnp.sum(diff * diff, axis=0)
            sch = jnp.sum(hfT * att1T[:, :, None], axis=0)
            return d2, sch

        d2h, sh = half_stats(hfT_h)
        d2a, sa = half_stats(hfT_a)
        dist = jnp.sqrt(jnp.concatenate([d2h, d2a], axis=0))
        y = gT - dist

        z = lax.bitcast_convert_type(y, jnp.int32)
        s = jnp.where(z < 0, z ^ jnp.int32(0x7FFFFFFF), z)
        lo = jnp.min(s, axis=0, keepdims=True)
        hi = jnp.max(s, axis=0, keepdims=True)
        one = jnp.int32(1)
        for _ in range(32):
            mid = (lo >> one) + (hi >> one) + (lo & hi & one)
            cnt = jnp.sum((s > mid).astype(jnp.int32), axis=0, keepdims=True)
            go = cnt >= K
            lo = jnp.where(go, mid + one, lo)
            hi = jnp.where(go, hi, mid)
        thr = lo
        gt = s > thr
        eq = s == thr
        n_gt = jnp.sum(gt.astype(jnp.int32), axis=0, keepdims=True)
        pre = tri @ eq.astype(jnp.float32)
        allowed = (K - n_gt).astype(jnp.float32)
        mask = jnp.logical_or(gt, jnp.logical_and(eq, pre <= allowed))

        s_rel = jnp.zeros_like(y)
        for r in range(7):
            srv = jnp.sum(r2e_ref[r : r + 1, :] * att2)
            s_rel = s_rel + jnp.where(rT == r, srv, 0.0)
        scT = jnp.concatenate([sh, sa], axis=0) + s_rel
        scm = jnp.where(mask, scT, -jnp.inf)
        mx = jnp.max(scm, axis=0, keepdims=True)
        p = jnp.where(mask, jnp.exp(scm - mx), 0.0)
        w = p * (1.0 / jnp.sum(p, axis=0, keepdims=True))

        embT = (jnp.sum(hfT_h * w[None, :L, :], axis=1)
                + jnp.sum(hfT_a * w[None, L:, :], axis=1))
        combT = (w1t_ref[...] @ jnp.concatenate([nfT, embT], axis=0)
                 + b1_ref[...])
        return jnp.maximum(combT, 0.0)

    aT = a_ref[...].T
    bT = b_ref[...].T

    def tr(ref):
        return ref[...].reshape(L * _BT, D).T.reshape(D, L, _BT)

    eu = encode(aT, bT, tr(huh_ref), tr(hua_ref), gu_ref[...], ru_ref[...])
    ev = encode(bT, aT, tr(hvh_ref), tr(hva_ref), gv_ref[...], rv_ref[...])
    out_ref[...] = jnp.sum(eu * ev, axis=0, keepdims=True)[None]


def _tc_scores(a, b, huh, hua, hvh, hva, guT, gvT, ruT, rvT,
               att, wqT, bqT, w1T, b1T, r2e, interpret=False):
    nb = B // _BT
    bspec = pl.BlockSpec
    grid_spec = pl.GridSpec(
        grid=(nb,),
        in_specs=[
            bspec((_BT, D), lambda i: (i, 0)),
            bspec((_BT, D), lambda i: (i, 0)),
            bspec((L, _BT, D), lambda i: (0, i, 0)),
            bspec((L, _BT, D), lambda i: (0, i, 0)),
            bspec((L, _BT, D), lambda i: (0, i, 0)),
            bspec((L, _BT, D), lambda i: (0, i, 0)),
            bspec((T, _BT), lambda i: (0, i)),
            bspec((T, _BT), lambda i: (0, i)),
            bspec((T, _BT), lambda i: (0, i)),
            bspec((T, _BT), lambda i: (0, i)),
            bspec((1, 2 * D), lambda i: (0, 0)),
            bspec((D, 2 * D), lambda i: (0, 0)),
            bspec((D, 1), lambda i: (0, 0)),
            bspec((D, 2 * D), lambda i: (0, 0)),
            bspec((D, 1), lambda i: (0, 0)),
            bspec((8, D), lambda i: (0, 0)),
        ],
        out_specs=bspec((1, 1, _BT), lambda i: (i, 0, 0)),
    )
    out = pl.pallas_call(
        _tc_body,
        grid_spec=grid_spec,
        out_shape=jax.ShapeDtypeStruct((nb, 1, _BT), jnp.float32),
        interpret=interpret,
    )(a, b, huh, hua, hvh, hva, guT, gvT, ruT, rvT,
      att, wqT, bqT, w1T, b1T, r2e)
    return out.reshape(B)


def _gumbel(key, shape):
    u = jax.random.uniform(key, shape)
    return -jnp.log(-jnp.log(u + 1e-10) + 1e-10)


_CONST = {}


def _gumbelsT():
    if not _CONST:
        gu = _gumbel(jax.random.fold_in(jax.random.key(42), 1), (B, T))
        gv = _gumbel(jax.random.fold_in(jax.random.key(42), 2), (B, T))
        _CONST["guT"] = jax.block_until_ready(gu.T)
        _CONST["gvT"] = jax.block_until_ready(gv.T)
    return _CONST["guT"], _CONST["gvT"]


def kernel(nodes_u, nodes_v, hist_u_n, hist_u_r, adj_u, hist_v_n, hist_v_r,
           adj_v, u2e, v2e, r2e, relation_att, Wq, bq, W1, b1):
    i32 = jnp.int32
    nodes_u = nodes_u.astype(i32)
    nodes_v = nodes_v.astype(i32)
    hist_u_n = hist_u_n.astype(i32)
    adj_u = adj_u.astype(i32)
    hist_v_n = hist_v_n.astype(i32)
    adj_v = adj_v.astype(i32)

    a, b, huh, hua, hvh, hva = _sc_gather(
        u2e, v2e, nodes_u, nodes_v, hist_u_n.T, adj_u.T, hist_v_n.T, adj_v.T)

    guT, gvT = _gumbelsT()
    six = jnp.full((M, B), 6, dtype=i32)
    ruT = jnp.concatenate([hist_u_r.astype(i32).T, six], axis=0)
    rvT = jnp.concatenate([hist_v_r.astype(i32).T, six], axis=0)
    r2e_p = jnp.concatenate([r2e, jnp.zeros((1, D), r2e.dtype)], axis=0)

    return _tc_scores(
        a, b,
        huh.reshape(L, B, D), hua.reshape(L, B, D),
        hvh.reshape(L, B, D), hva.reshape(L, B, D),
        guT, gvT, ruT, rvT,
        relation_att.reshape(1, 2 * D), Wq.T, bq.reshape(D, 1),
        W1.T, b1.reshape(D, 1), r2e_p)
